# baseline (speedup 1.0000x reference)
import functools

import jax
import jax.numpy as jnp
from jax import lax
from jax.experimental import pallas as pl
from jax.experimental.pallas import tpu as pltpu
from jax.experimental.pallas import tpu_sc as plsc

N = 10000
E = 320000
D = 128
NPAD = 10240
NW = 32
CHUNK = 128
TOT_CH = 2560
EPAD = TOT_CH * CHUNK
CH0 = 118
CH1 = 42
DCH = TOT_CH // NW
RING = 8
ROWS_PER_TILE = NPAD // 16

_mesh = plsc.VectorSubcoreMesh(core_axis_name="c", subcore_axis_name="s")


def _zero_buf(buf):
    rows, cols = buf.shape

    @pl.loop(0, rows)
    def _(j):
        @pl.loop(0, cols // 16)
        def _(k):
            buf[j, pl.ds(k * 16, 16)] = jnp.zeros((16,), jnp.float32)


@functools.partial(
    pl.kernel,
    out_type=jax.ShapeDtypeStruct((2, NPAD, 16), jnp.float32),
    mesh=_mesh,
    scratch_types=[
        pltpu.VMEM((DCH, CHUNK), jnp.int32),
        pltpu.VMEM((CHUNK, 16), jnp.float32),
        pltpu.VMEM((CHUNK, 16), jnp.float32),
        pltpu.VMEM_SHARED((NPAD, 16), jnp.float32),
        pltpu.SemaphoreType.DMA,
    ],
)
def _deg_kernel(dst_hbm, out_hbm, dst_v, ones_v, z_v, acc, sem):
    c = lax.axis_index("c")
    s = lax.axis_index("s")
    wid = c * 16 + s
    pltpu.sync_copy(dst_hbm.at[wid], dst_v)
    _zero_buf(z_v)

    @pl.loop(0, CHUNK)
    def _(j):
        ones_v[j, pl.ds(0, 16)] = jnp.ones((16,), jnp.float32)

    @pl.loop(0, ROWS_PER_TILE // CHUNK)
    def _(j):
        pltpu.sync_copy(z_v, acc.at[pl.ds(s * ROWS_PER_TILE + j * CHUNK, CHUNK)])

    plsc.subcore_barrier()

    @pl.loop(0, DCH)
    def _(i):
        pltpu.sync_copy(ones_v, acc.at[dst_v.at[i]], add=True)

    plsc.subcore_barrier()

    @pl.loop(0, ROWS_PER_TILE // CHUNK)
    def _(j):
        r = s * ROWS_PER_TILE + j * CHUNK
        pltpu.sync_copy(acc.at[pl.ds(r, CHUNK)], out_hbm.at[c, pl.ds(r, CHUNK)])


@functools.partial(
    pl.kernel,
    out_type=jax.ShapeDtypeStruct((2, NPAD, D), jnp.float32),
    mesh=_mesh,
    scratch_types=[
        pltpu.VMEM((2 * RING, CHUNK), jnp.int32),
        pltpu.VMEM((CHUNK, D), jnp.float32),
        pltpu.VMEM((CHUNK, D), jnp.float32),
        pltpu.VMEM_SHARED((NPAD, D), jnp.float32),
        pltpu.SemaphoreType.DMA,
        pltpu.SemaphoreType.DMA,
        pltpu.SemaphoreType.DMA,
    ],
)
def _agg_kernel(g_hbm, idx_hbm, out_hbm,
                ring, rows_a, rows_b, acc, sem_i, sem_a, sem_b):
    c = lax.axis_index("c")
    s = lax.axis_index("s")
    base = jnp.where(c == 0, s * CH0, 16 * CH0 + s * CH1)
    nch = jnp.where(c == 0, CH0, CH1)

    _zero_buf(rows_a)

    @pl.loop(0, ROWS_PER_TILE // CHUNK)
    def _(j):
        pltpu.sync_copy(rows_a, acc.at[pl.ds(s * ROWS_PER_TILE + j * CHUNK, CHUNK)])

    pltpu.sync_copy(idx_hbm.at[base], ring.at[pl.ds(0, 2)])
    pltpu.sync_copy(idx_hbm.at[base + 1], ring.at[pl.ds(2, 2)])
    plsc.subcore_barrier()
    pltpu.async_copy(g_hbm.at[ring.at[0]], rows_a, sem_a)

    @pl.loop(0, CH0)
    def _(i):
        @pl.when(i < nch)
        def _():
            nxt = i + 2

            @pl.when((i >= 1) & (i + 1 < nch))
            def _():
                pltpu.make_async_copy(idx_hbm.at[base],
                                      ring.at[pl.ds(0, 2)], sem_i).wait()

            @pl.when(nxt < nch)
            def _():
                pltpu.async_copy(idx_hbm.at[base + nxt],
                                 ring.at[pl.ds(2 * (nxt % RING), 2)], sem_i)

            even = i % 2 == 0
            g_row = 2 * ((i + 1) % RING)
            s_row = 2 * (i % RING) + 1

            @pl.when(even)
            def _():
                @pl.when(i + 1 < nch)
                def _():
                    pltpu.async_copy(g_hbm.at[ring.at[g_row]], rows_b, sem_b)
                pltpu.make_async_copy(g_hbm.at[ring.at[g_row]], rows_a,
                                      sem_a).wait()
                pltpu.sync_copy(rows_a, acc.at[ring.at[s_row]], add=True)

            @pl.when(jnp.logical_not(even))
            def _():
                @pl.when(i + 1 < nch)
                def _():
                    pltpu.async_copy(g_hbm.at[ring.at[g_row]], rows_a, sem_a)
                pltpu.make_async_copy(g_hbm.at[ring.at[g_row]], rows_b,
                                      sem_b).wait()
                pltpu.sync_copy(rows_b, acc.at[ring.at[s_row]], add=True)

    plsc.subcore_barrier()

    @pl.loop(0, ROWS_PER_TILE // CHUNK)
    def _(j):
        r = s * ROWS_PER_TILE + j * CHUNK
        pltpu.sync_copy(acc.at[pl.ds(r, CHUNK)], out_hbm.at[c, pl.ds(r, CHUNK)])


def _mm_body(x_ref, w_ref, o_ref):
    o_ref[...] = jnp.dot(x_ref[...], w_ref[...],
                         preferred_element_type=jnp.float32)


def _tc_matmul(x, w):
    return pl.pallas_call(
        _mm_body,
        out_shape=jax.ShapeDtypeStruct((x.shape[0], w.shape[1]), jnp.float32),
    )(x, w)


def _scale_body(d0_ref, d1_ref, xw_ref, g_ref, dinv_ref):
    deg = d0_ref[:, :1] + d1_ref[:, :1] + 1.0
    dinv = lax.rsqrt(deg)
    g_ref[...] = dinv * xw_ref[...]
    dinv_ref[...] = dinv


def _tc_scale(deg0, deg1, xw):
    return pl.pallas_call(
        _scale_body,
        out_shape=(jax.ShapeDtypeStruct((NPAD, D), jnp.float32),
                   jax.ShapeDtypeStruct((NPAD, 1), jnp.float32)),
    )(deg0, deg1, xw)


def _mid_body(p0_ref, p1_ref, g_ref, dinv_ref, b_ref, w_ref, gout_ref):
    h = jnp.maximum(
        dinv_ref[...] * (p0_ref[...] + p1_ref[...] + g_ref[...]) + b_ref[...],
        0.0)
    gout_ref[...] = dinv_ref[...] * jnp.dot(h, w_ref[...],
                                            preferred_element_type=jnp.float32)


def _tc_mid(p0, p1, g, dinv, b, w):
    return pl.pallas_call(
        _mid_body,
        out_shape=jax.ShapeDtypeStruct((NPAD, D), jnp.float32),
    )(p0, p1, g, dinv, b, w)


def _fin_body(p0_ref, p1_ref, g_ref, dinv_ref, b_ref, o_ref):
    o_ref[...] = (dinv_ref[...] * (p0_ref[...] + p1_ref[...] + g_ref[...])
                  + b_ref[...])


def _tc_fin(p0, p1, g, dinv, b):
    return pl.pallas_call(
        _fin_body,
        out_shape=jax.ShapeDtypeStruct((NPAD, D), jnp.float32),
    )(p0, p1, g, dinv, b)


def kernel(x, edge_index, W1, b1, W2, b2, W3, b3):
    src = edge_index[0].astype(jnp.int32)
    dst = edge_index[1].astype(jnp.int32)
    pad = jnp.full((EPAD - E,), NPAD - 1, dtype=jnp.int32)
    src_p = jnp.concatenate([src, pad])
    dst_p = jnp.concatenate([dst, pad])
    dst_r = dst_p.reshape(NW, DCH, CHUNK)
    idx_comb = jnp.stack(
        [src_p.reshape(TOT_CH, CHUNK), dst_p.reshape(TOT_CH, CHUNK)], axis=1)
    x_pad = jnp.pad(x, ((0, NPAD - N), (0, 0)))

    degp = _deg_kernel(dst_r)
    xw1 = _tc_matmul(x_pad, W1)
    g1, dinv = _tc_scale(degp[0], degp[1], xw1)

    b1r = b1.reshape(1, D)
    b2r = b2.reshape(1, D)
    b3r = b3.reshape(1, D)

    p = _agg_kernel(g1, idx_comb)
    g2 = _tc_mid(p[0], p[1], g1, dinv, b1r, W2)
    p = _agg_kernel(g2, idx_comb)
    g3 = _tc_mid(p[0], p[1], g2, dinv, b2r, W3)
    p = _agg_kernel(g3, idx_comb)
    out = _tc_fin(p[0], p[1], g3, dinv, b3r)
    return out[:N]


# speedup vs baseline: 3.1933x; 3.1933x over previous
import functools

import jax
import jax.numpy as jnp
from jax import lax
from jax.experimental import pallas as pl
from jax.experimental.pallas import tpu as pltpu
from jax.experimental.pallas import tpu_sc as plsc

N = 10000
E = 320000
D = 128
NPAD = 10240
NW = 32
CHUNK = 128
TOT_CH = 2528
EPAD = TOT_CH * CHUNK
CH0 = 79
CH1 = 79
DCH = TOT_CH // NW
RING = 8
ROWS_PER_TILE = NPAD // 16

_mesh = plsc.VectorSubcoreMesh(core_axis_name="c", subcore_axis_name="s")


def _zero_buf(buf):
    rows, cols = buf.shape

    @pl.loop(0, rows)
    def _(j):
        @pl.loop(0, cols // 16)
        def _(k):
            buf[j, pl.ds(k * 16, 16)] = jnp.zeros((16,), jnp.float32)


@functools.partial(
    pl.kernel,
    out_type=jax.ShapeDtypeStruct((2, NPAD, 16), jnp.float32),
    mesh=_mesh,
    scratch_types=[
        pltpu.VMEM((DCH, CHUNK), jnp.int32),
        pltpu.VMEM((CHUNK, 16), jnp.float32),
        pltpu.VMEM((CHUNK, 16), jnp.float32),
        pltpu.VMEM_SHARED((NPAD, 16), jnp.float32),
        pltpu.SemaphoreType.DMA,
    ],
)
def _deg_kernel(dst_hbm, out_hbm, dst_v, ones_v, z_v, acc, sem):
    c = lax.axis_index("c")
    s = lax.axis_index("s")
    wid = c * 16 + s
    pltpu.sync_copy(dst_hbm.at[wid], dst_v)
    _zero_buf(z_v)

    @pl.loop(0, CHUNK)
    def _(j):
        ones_v[j, pl.ds(0, 16)] = jnp.ones((16,), jnp.float32)

    @pl.loop(0, ROWS_PER_TILE // CHUNK)
    def _(j):
        pltpu.sync_copy(z_v, acc.at[pl.ds(s * ROWS_PER_TILE + j * CHUNK, CHUNK)])

    plsc.subcore_barrier()

    @pl.loop(0, DCH)
    def _(i):
        pltpu.sync_copy(ones_v, acc.at[dst_v.at[i]], add=True)

    plsc.subcore_barrier()

    @pl.loop(0, ROWS_PER_TILE // CHUNK)
    def _(j):
        r = s * ROWS_PER_TILE + j * CHUNK
        pltpu.sync_copy(acc.at[pl.ds(r, CHUNK)], out_hbm.at[c, pl.ds(r, CHUNK)])


@functools.partial(
    pl.kernel,
    out_type=jax.ShapeDtypeStruct((2, NPAD, D), jnp.float32),
    mesh=_mesh,
    scratch_types=[
        pltpu.VMEM((2 * RING, CHUNK), jnp.int32),
        pltpu.VMEM((CHUNK, D), jnp.float32),
        pltpu.VMEM((CHUNK, D), jnp.float32),
        pltpu.VMEM_SHARED((NPAD, D), jnp.float32),
        pltpu.SemaphoreType.DMA,
        pltpu.SemaphoreType.DMA,
        pltpu.SemaphoreType.DMA,
    ],
)
def _agg_kernel(g_hbm, idx_hbm, out_hbm,
                ring, rows_a, rows_b, acc, sem_i, sem_a, sem_b):
    c = lax.axis_index("c")
    s = lax.axis_index("s")
    base = jnp.where(c == 0, s * CH0, 16 * CH0 + s * CH1)
    nch = jnp.where(c == 0, CH0, CH1)

    _zero_buf(rows_a)

    @pl.loop(0, ROWS_PER_TILE // CHUNK)
    def _(j):
        pltpu.sync_copy(rows_a, acc.at[pl.ds(s * ROWS_PER_TILE + j * CHUNK, CHUNK)])

    pltpu.sync_copy(idx_hbm.at[base], ring.at[pl.ds(0, 2)])
    pltpu.sync_copy(idx_hbm.at[base + 1], ring.at[pl.ds(2, 2)])
    plsc.subcore_barrier()
    pltpu.async_copy(g_hbm.at[ring.at[0]], rows_a, sem_a)

    @pl.loop(0, CH0)
    def _(i):
        @pl.when(i < nch)
        def _():
            nxt = i + 2

            @pl.when((i >= 1) & (i + 1 < nch))
            def _():
                pltpu.make_async_copy(idx_hbm.at[base],
                                      ring.at[pl.ds(0, 2)], sem_i).wait()

            @pl.when(nxt < nch)
            def _():
                pltpu.async_copy(idx_hbm.at[base + nxt],
                                 ring.at[pl.ds(2 * (nxt % RING), 2)], sem_i)

            even = i % 2 == 0
            g_row = 2 * ((i + 1) % RING)
            s_row = 2 * (i % RING) + 1

            @pl.when(even)
            def _():
                @pl.when(i + 1 < nch)
                def _():
                    pltpu.async_copy(g_hbm.at[ring.at[g_row]], rows_b, sem_b)
                pltpu.make_async_copy(g_hbm.at[ring.at[g_row]], rows_a,
                                      sem_a).wait()
                pltpu.sync_copy(rows_a, acc.at[ring.at[s_row]], add=True)

            @pl.when(jnp.logical_not(even))
            def _():
                @pl.when(i + 1 < nch)
                def _():
                    pltpu.async_copy(g_hbm.at[ring.at[g_row]], rows_a, sem_a)
                pltpu.make_async_copy(g_hbm.at[ring.at[g_row]], rows_b,
                                      sem_b).wait()
                pltpu.sync_copy(rows_b, acc.at[ring.at[s_row]], add=True)

    plsc.subcore_barrier()

    @pl.loop(0, ROWS_PER_TILE // CHUNK)
    def _(j):
        r = s * ROWS_PER_TILE + j * CHUNK
        pltpu.sync_copy(acc.at[pl.ds(r, CHUNK)], out_hbm.at[c, pl.ds(r, CHUNK)])


def _mm_body(x_ref, w_ref, o_ref):
    o_ref[...] = jnp.dot(x_ref[...], w_ref[...],
                         preferred_element_type=jnp.float32)


def _tc_matmul(x, w):
    return pl.pallas_call(
        _mm_body,
        out_shape=jax.ShapeDtypeStruct((x.shape[0], w.shape[1]), jnp.float32),
    )(x, w)


def _scale_body(d0_ref, d1_ref, xw_ref, g_ref, dinv_ref):
    deg = d0_ref[:, :1] + d1_ref[:, :1] + 1.0
    dinv = lax.rsqrt(deg)
    g_ref[...] = dinv * xw_ref[...]
    dinv_ref[...] = dinv


def _tc_scale(deg0, deg1, xw):
    return pl.pallas_call(
        _scale_body,
        out_shape=(jax.ShapeDtypeStruct((NPAD, D), jnp.float32),
                   jax.ShapeDtypeStruct((NPAD, 1), jnp.float32)),
    )(deg0, deg1, xw)


def _mid_body(p0_ref, p1_ref, g_ref, dinv_ref, b_ref, w_ref, gout_ref):
    h = jnp.maximum(
        dinv_ref[...] * (p0_ref[...] + p1_ref[...] + g_ref[...]) + b_ref[...],
        0.0)
    gout_ref[...] = dinv_ref[...] * jnp.dot(h, w_ref[...],
                                            preferred_element_type=jnp.float32)


def _tc_mid(p0, p1, g, dinv, b, w):
    return pl.pallas_call(
        _mid_body,
        out_shape=jax.ShapeDtypeStruct((NPAD, D), jnp.float32),
    )(p0, p1, g, dinv, b, w)


def _fin_body(p0_ref, p1_ref, g_ref, dinv_ref, b_ref, o_ref):
    o_ref[...] = (dinv_ref[...] * (p0_ref[...] + p1_ref[...] + g_ref[...])
                  + b_ref[...])


def _tc_fin(p0, p1, g, dinv, b):
    return pl.pallas_call(
        _fin_body,
        out_shape=jax.ShapeDtypeStruct((NPAD, D), jnp.float32),
    )(p0, p1, g, dinv, b)


def kernel(x, edge_index, W1, b1, W2, b2, W3, b3):
    src = edge_index[0].astype(jnp.int32)
    dst = edge_index[1].astype(jnp.int32)
    pad = N + (jnp.arange(EPAD - E, dtype=jnp.int32) % (NPAD - N))
    src_p = jnp.concatenate([src, pad])
    dst_p = jnp.concatenate([dst, pad])
    dst_r = dst_p.reshape(NW, DCH, CHUNK)
    idx_comb = jnp.stack(
        [src_p.reshape(TOT_CH, CHUNK), dst_p.reshape(TOT_CH, CHUNK)], axis=1)
    x_pad = jnp.pad(x, ((0, NPAD - N), (0, 0)))

    degp = _deg_kernel(dst_r)
    xw1 = _tc_matmul(x_pad, W1)
    g1, dinv = _tc_scale(degp[0], degp[1], xw1)

    b1r = b1.reshape(1, D)
    b2r = b2.reshape(1, D)
    b3r = b3.reshape(1, D)

    p = _agg_kernel(g1, idx_comb)
    g2 = _tc_mid(p[0], p[1], g1, dinv, b1r, W2)
    p = _agg_kernel(g2, idx_comb)
    g3 = _tc_mid(p[0], p[1], g2, dinv, b2r, W3)
    p = _agg_kernel(g3, idx_comb)
    out = _tc_fin(p[0], p[1], g3, dinv, b3r)
    return out[:N]


# speedup vs baseline: 3.4237x; 1.0721x over previous
import functools

import jax
import jax.numpy as jnp
from jax import lax
from jax.experimental import pallas as pl
from jax.experimental.pallas import tpu as pltpu
from jax.experimental.pallas import tpu_sc as plsc

N = 10000
E = 320000
D = 128
NPAD = 10240
NW = 32
CHUNK = 128
TOT_CH = E // CHUNK
CHMIN = 78
XWID = NW - (TOT_CH - CHMIN * NW)
CHMAX = CHMIN + 1
DCH = 79
DPAD = NW * DCH * CHUNK - E
RING = 8
ROWS_PER_TILE = NPAD // 16

_mesh = plsc.VectorSubcoreMesh(core_axis_name="c", subcore_axis_name="s")


def _zero_buf(buf):
    rows, cols = buf.shape

    @pl.loop(0, rows)
    def _(j):
        @pl.loop(0, cols // 16)
        def _(k):
            buf[j, pl.ds(k * 16, 16)] = jnp.zeros((16,), jnp.float32)


@functools.partial(
    pl.kernel,
    out_type=jax.ShapeDtypeStruct((2, NPAD, 16), jnp.float32),
    mesh=_mesh,
    scratch_types=[
        pltpu.VMEM((DCH, CHUNK), jnp.int32),
        pltpu.VMEM((CHUNK, 16), jnp.float32),
        pltpu.VMEM((CHUNK, 16), jnp.float32),
        pltpu.VMEM_SHARED((NPAD, 16), jnp.float32),
        pltpu.SemaphoreType.DMA,
    ],
)
def _deg_kernel(dst_hbm, out_hbm, dst_v, ones_v, z_v, acc, sem):
    c = lax.axis_index("c")
    s = lax.axis_index("s")
    wid = c * 16 + s
    pltpu.sync_copy(dst_hbm.at[wid], dst_v)
    _zero_buf(z_v)

    @pl.loop(0, CHUNK)
    def _(j):
        ones_v[j, pl.ds(0, 16)] = jnp.ones((16,), jnp.float32)

    @pl.loop(0, ROWS_PER_TILE // CHUNK)
    def _(j):
        pltpu.sync_copy(z_v, acc.at[pl.ds(s * ROWS_PER_TILE + j * CHUNK, CHUNK)])

    plsc.subcore_barrier()

    @pl.loop(0, DCH)
    def _(i):
        pltpu.sync_copy(ones_v, acc.at[dst_v.at[i]], add=True)

    plsc.subcore_barrier()

    @pl.loop(0, ROWS_PER_TILE // CHUNK)
    def _(j):
        r = s * ROWS_PER_TILE + j * CHUNK
        pltpu.sync_copy(acc.at[pl.ds(r, CHUNK)], out_hbm.at[c, pl.ds(r, CHUNK)])


@functools.partial(
    pl.kernel,
    out_type=jax.ShapeDtypeStruct((2, NPAD, D), jnp.float32),
    mesh=_mesh,
    scratch_types=[
        pltpu.VMEM((RING, CHUNK), jnp.int32),
        pltpu.VMEM((RING, CHUNK), jnp.int32),
        pltpu.VMEM((CHUNK, D), jnp.float32),
        pltpu.VMEM((CHUNK, D), jnp.float32),
        pltpu.VMEM_SHARED((NPAD, D), jnp.float32),
        pltpu.SemaphoreType.DMA,
        pltpu.SemaphoreType.DMA,
        pltpu.SemaphoreType.DMA,
    ],
)
def _agg_kernel(g_hbm, src_hbm, dst_hbm, out_hbm,
                ring_s, ring_d, rows_a, rows_b, acc, sem_i, sem_a, sem_b):
    c = lax.axis_index("c")
    s = lax.axis_index("s")
    wid = c * 16 + s
    base = wid * CHMIN + jnp.maximum(wid - XWID, 0)
    nch = jnp.where(wid >= XWID, CHMAX, CHMIN)

    _zero_buf(rows_a)

    @pl.loop(0, ROWS_PER_TILE // CHUNK)
    def _(j):
        pltpu.sync_copy(rows_a, acc.at[pl.ds(s * ROWS_PER_TILE + j * CHUNK, CHUNK)])

    pltpu.sync_copy(src_hbm.at[base], ring_s.at[0])
    pltpu.sync_copy(dst_hbm.at[base], ring_d.at[0])
    pltpu.sync_copy(src_hbm.at[base + 1], ring_s.at[1])
    pltpu.sync_copy(dst_hbm.at[base + 1], ring_d.at[1])
    plsc.subcore_barrier()
    pltpu.async_copy(g_hbm.at[ring_s.at[0]], rows_a, sem_a)

    @pl.loop(0, CHMAX)
    def _(i):
        @pl.when(i < nch)
        def _():
            nxt = i + 2

            @pl.when((i >= 1) & (i + 1 < nch))
            def _():
                pltpu.make_async_copy(src_hbm.at[base], ring_s.at[0],
                                      sem_i).wait()
                pltpu.make_async_copy(dst_hbm.at[base], ring_d.at[0],
                                      sem_i).wait()

            @pl.when(nxt < nch)
            def _():
                pltpu.async_copy(src_hbm.at[base + nxt],
                                 ring_s.at[nxt % RING], sem_i)
                pltpu.async_copy(dst_hbm.at[base + nxt],
                                 ring_d.at[nxt % RING], sem_i)

            even = i % 2 == 0
            g_row = (i + 1) % RING
            s_row = i % RING

            @pl.when(even)
            def _():
                @pl.when(i + 1 < nch)
                def _():
                    pltpu.async_copy(g_hbm.at[ring_s.at[g_row]], rows_b, sem_b)
                pltpu.make_async_copy(g_hbm.at[ring_s.at[g_row]], rows_a,
                                      sem_a).wait()
                pltpu.sync_copy(rows_a, acc.at[ring_d.at[s_row]], add=True)

            @pl.when(jnp.logical_not(even))
            def _():
                @pl.when(i + 1 < nch)
                def _():
                    pltpu.async_copy(g_hbm.at[ring_s.at[g_row]], rows_a, sem_a)
                pltpu.make_async_copy(g_hbm.at[ring_s.at[g_row]], rows_b,
                                      sem_b).wait()
                pltpu.sync_copy(rows_b, acc.at[ring_d.at[s_row]], add=True)

    plsc.subcore_barrier()

    @pl.loop(0, ROWS_PER_TILE // CHUNK)
    def _(j):
        r = s * ROWS_PER_TILE + j * CHUNK
        pltpu.sync_copy(acc.at[pl.ds(r, CHUNK)], out_hbm.at[c, pl.ds(r, CHUNK)])


def _mm_body(x_ref, w_ref, o_ref):
    o_ref[...] = jnp.dot(x_ref[...], w_ref[...],
                         preferred_element_type=jnp.float32)


def _tc_matmul(x, w):
    return pl.pallas_call(
        _mm_body,
        out_shape=jax.ShapeDtypeStruct((x.shape[0], w.shape[1]), jnp.float32),
    )(x, w)


def _scale_body(dp_ref, xw_ref, g_ref, dinv_ref):
    deg = dp_ref[0, :, :1] + dp_ref[1, :, :1] + 1.0
    dinv = lax.rsqrt(deg)
    g_ref[...] = dinv * xw_ref[...]
    dinv_ref[...] = dinv


def _tc_scale(degp, xw):
    return pl.pallas_call(
        _scale_body,
        out_shape=(jax.ShapeDtypeStruct((NPAD, D), jnp.float32),
                   jax.ShapeDtypeStruct((NPAD, 1), jnp.float32)),
    )(degp, xw)


def _mid_body(p_ref, g_ref, dinv_ref, b_ref, w_ref, gout_ref):
    h = jnp.maximum(
        dinv_ref[...] * (p_ref[0] + p_ref[1] + g_ref[...]) + b_ref[...],
        0.0)
    gout_ref[...] = dinv_ref[...] * jnp.dot(h, w_ref[...],
                                            preferred_element_type=jnp.float32)


def _tc_mid(p, g, dinv, b, w):
    return pl.pallas_call(
        _mid_body,
        out_shape=jax.ShapeDtypeStruct((NPAD, D), jnp.float32),
    )(p, g, dinv, b, w)


def _fin_body(p_ref, g_ref, dinv_ref, b_ref, o_ref):
    o_ref[...] = (dinv_ref[...] * (p_ref[0] + p_ref[1] + g_ref[...])
                  + b_ref[...])


def _tc_fin(p, g, dinv, b):
    return pl.pallas_call(
        _fin_body,
        out_shape=jax.ShapeDtypeStruct((NPAD, D), jnp.float32),
    )(p, g, dinv, b)


def kernel(x, edge_index, W1, b1, W2, b2, W3, b3):
    src2d = edge_index[0].astype(jnp.int32).reshape(TOT_CH, CHUNK)
    dst2d = edge_index[1].astype(jnp.int32).reshape(TOT_CH, CHUNK)
    x_pad = jnp.pad(x, ((0, NPAD - N), (0, 0)))
    dpad = N + (jnp.arange(DPAD, dtype=jnp.int32) % (NPAD - N))
    dst_deg = jnp.concatenate([edge_index[1].astype(jnp.int32), dpad]
                              ).reshape(NW, DCH, CHUNK)

    degp = _deg_kernel(dst_deg)
    xw1 = _tc_matmul(x_pad, W1)
    g1, dinv = _tc_scale(degp, xw1)

    b1r = b1.reshape(1, D)
    b2r = b2.reshape(1, D)
    b3r = b3.reshape(1, D)

    p = _agg_kernel(g1, src2d, dst2d)
    g2 = _tc_mid(p, g1, dinv, b1r, W2)
    p = _agg_kernel(g2, src2d, dst2d)
    g3 = _tc_mid(p, g2, dinv, b2r, W3)
    p = _agg_kernel(g3, src2d, dst2d)
    out = _tc_fin(p, g3, dinv, b3r)
    return out[:N]
